# baseline (speedup 1.0000x reference)
import functools

import jax
import jax.numpy as jnp
from jax import lax
from jax.experimental import pallas as pl
from jax.experimental.pallas import tpu as pltpu
from jax.experimental.pallas import tpu_sc as plsc

N = 10000
E = 320000
NF = 128
NH = 256
ND = 128
F = 64
NC = 2
NS = 16
LANES = 16
B = 128
NB = 160
EPT = NB * B
E_PAD = NS * EPT
NPAD = 10240
NSLOT = 4
import numpy as np

MAGIC = np.int32(0x5F3759DF)
CMASK = np.int32(16383)


def _rsqrt16(v):
    i = plsc.bitcast(v, jnp.int32)
    y = plsc.bitcast(MAGIC - (i >> 1), jnp.float32)
    for _ in range(3):
        y = y * (1.5 - 0.5 * v * y * y)
    return jnp.where(v > 0.0, y, 0.0)


def _zero16():
    return jnp.zeros((LANES,), jnp.float32)


def _sc_body(x0, x1, rcb, wb,
             h1a, h1b, h2a, h2b,
             rc_ref, w_ref, dd_ref, abuf, tbuf, rbuf, ridx, cidx,
             dis_sh, h_acc, gsem, ssem):
    c = lax.axis_index("c")
    t = lax.axis_index("s")

    pltpu.sync_copy(rcb.at[t], rc_ref)
    pltpu.sync_copy(wb.at[t], w_ref)

    @pl.loop(0, 160)
    def _(r):
        for j in range(F // LANES):
            dd_ref[r, pl.ds(j * LANES, LANES)] = _zero16()

    for r in range(10):
        for j in range(F // LANES):
            abuf[r, pl.ds(j * LANES, LANES)] = _zero16()

    @pl.loop(0, NB)
    def _(b):
        for k in range(B // LANES):
            sl = pl.ds(k * LANES, LANES)
            cv = rc_ref[b, sl] & CMASK
            plsc.addupdate_scatter(dd_ref, [cv >> 6, cv & 63], w_ref[b, sl])

    pltpu.sync_copy(dd_ref, h_acc.at[pl.ds(160 * t, 160)])
    plsc.subcore_barrier()

    for j in range(NS):
        pltpu.sync_copy(h_acc.at[pl.ds(j * 160 + 10 * t, 10)], tbuf)
        for r in range(10):
            for jj in range(F // LANES):
                sl = pl.ds(jj * LANES, LANES)
                abuf[r, sl] = abuf[r, sl] + tbuf[r, sl]
    for r in range(10):
        for jj in range(F // LANES):
            sl = pl.ds(jj * LANES, LANES)
            abuf[r, sl] = _rsqrt16(abuf[r, sl])
    pltpu.sync_copy(abuf, dis_sh.at[pl.ds(10 * t, 10)])
    plsc.subcore_barrier()
    pltpu.sync_copy(dis_sh, dd_ref)

    @pl.loop(0, NB)
    def _(b):
        for k in range(B // LANES):
            sl = pl.ds(k * LANES, LANES)
            pk = rc_ref[b, sl]
            rv = pk >> 14
            cv = pk & CMASK
            wv = w_ref[b, sl]
            dr = plsc.load_gather(dd_ref, [rv >> 6, rv & 63])
            dc = plsc.load_gather(dd_ref, [cv >> 6, cv & 63])
            w_ref[b, sl] = dr * wv * dc

    def unpack(b, s):
        for k in range(B // LANES):
            sl = pl.ds(k * LANES, LANES)
            pk = rc_ref[b, sl]
            ridx[s, sl] = pk >> 14
            cidx[s, sl] = pk & CMASK

    def hop(src_hbm, dst_hbm):
        plsc.subcore_barrier()

        z0 = 3 * B

        @pl.loop(0, B)
        def _(r):
            for j in range(F // LANES):
                rbuf[z0 + r, pl.ds(j * LANES, LANES)] = _zero16()

        for z in range(5):
            pltpu.sync_copy(rbuf.at[pl.ds(z0, B)],
                            h_acc.at[pl.ds(t * 640 + z * B, B)])
        plsc.subcore_barrier()

        def scale(s, b):
            buf = rbuf.at[pl.ds(s * B, B)]

            @pl.loop(0, B // LANES)
            def _(g):
                nv = w_ref[b, pl.ds(g * LANES, LANES)]
                for i in range(LANES):
                    sv = nv[i]
                    e = g * LANES + i
                    for j in range(F // LANES):
                        sl = pl.ds(j * LANES, LANES)
                        buf[e, sl] = buf[e, sl] * sv

        for b0 in range(2):
            unpack(b0, b0)
            pltpu.async_copy(src_hbm.at[ridx.at[b0]],
                             rbuf.at[pl.ds(b0 * B, B)], gsem.at[b0])

        @pl.loop(0, NB)
        def _(b):
            s = lax.rem(b, NSLOT)
            pltpu.make_async_copy(src_hbm.at[pl.ds(0, B)],
                                  rbuf.at[pl.ds(0, B)], gsem.at[s]).wait()
            scale(s, b)
            pltpu.async_copy(rbuf.at[pl.ds(s * B, B)],
                             h_acc.at[cidx.at[s]], ssem.at[s], add=True)

            @pl.when(b + 2 < NB)
            def _():
                s2 = lax.rem(b + 2, NSLOT)

                @pl.when(b >= 2)
                def _():
                    pltpu.make_async_copy(
                        rbuf.at[pl.ds(0, B)], h_acc.at[pl.ds(0, B)],
                        ssem.at[s2]).wait()

                unpack(b + 2, s2)
                pltpu.async_copy(src_hbm.at[ridx.at[s2]],
                                 rbuf.at[pl.ds(s2 * B, B)], gsem.at[s2])

        for b0 in range(NB - 4, NB):
            pltpu.make_async_copy(rbuf.at[pl.ds(0, B)],
                                  h_acc.at[pl.ds(0, B)],
                                  ssem.at[b0 % NSLOT]).wait()

        plsc.subcore_barrier()
        for z in range(5):
            base = t * 640 + z * B
            pltpu.sync_copy(h_acc.at[pl.ds(base, B)], rbuf.at[pl.ds(0, B)])
            pltpu.sync_copy(rbuf.at[pl.ds(0, B)], dst_hbm.at[pl.ds(base, B)])
        plsc.subcore_barrier()

    @pl.when(c == 0)
    def _():
        hop(x0, h1a)
        hop(h1a, h2a)

    @pl.when(c == 1)
    def _():
        hop(x1, h1b)
        hop(h1b, h2b)


_sc_prop = pl.kernel(
    _sc_body,
    out_type=[jax.ShapeDtypeStruct((NPAD, F), jnp.float32)] * 4,
    mesh=plsc.VectorSubcoreMesh(
        core_axis_name="c", subcore_axis_name="s", num_cores=NC,
        num_subcores=NS),
    compiler_params=pltpu.CompilerParams(
        needs_layout_passes=False, use_tc_tiling_on_sc=False),
    scratch_types=[
        pltpu.VMEM((NB, B), jnp.int32),
        pltpu.VMEM((NB, B), jnp.float32),
        pltpu.VMEM((160, F), jnp.float32),
        pltpu.VMEM((10, F), jnp.float32),
        pltpu.VMEM((10, F), jnp.float32),
        pltpu.VMEM((NSLOT * B, F), jnp.float32),
        pltpu.VMEM((NSLOT, B), jnp.int32),
        pltpu.VMEM((NSLOT, B), jnp.int32),
        pltpu.VMEM_SHARED((160, F), jnp.float32),
        pltpu.VMEM_SHARED((NPAD, F), jnp.float32),
        pltpu.SemaphoreType.DMA((NSLOT,)),
        pltpu.SemaphoreType.DMA((NSLOT,)),
    ],
)


def _mlp_body(h2a_ref, h2b_ref, wg0_ref, wg1_ref, w1t_ref, b1_ref, w2t_ref,
              y_ref, emb_ref):
    a = h2a_ref[...]
    b = h2b_ref[...]
    emb = (jnp.dot(a, wg0_ref[...], preferred_element_type=jnp.float32)
           + jnp.dot(b, wg1_ref[...], preferred_element_type=jnp.float32))
    emb_ref[...] = emb
    u = jnp.maximum(emb, 0.0)
    u = jnp.dot(u, w1t_ref[...], preferred_element_type=jnp.float32)
    u = jnp.maximum(u + b1_ref[...], 0.0)
    y_ref[...] = jnp.dot(u, w2t_ref[...], preferred_element_type=jnp.float32)


_RB = 1000


def _mlp(h2a, h2b, wg0, wg1, w1t, b1r, w2t):
    return pl.pallas_call(
        _mlp_body,
        grid=(N // _RB,),
        in_specs=[
            pl.BlockSpec((_RB, F), lambda i: (i, 0)),
            pl.BlockSpec((_RB, F), lambda i: (i, 0)),
            pl.BlockSpec((F, ND), lambda i: (0, 0)),
            pl.BlockSpec((F, ND), lambda i: (0, 0)),
            pl.BlockSpec((ND, NH), lambda i: (0, 0)),
            pl.BlockSpec((1, NH), lambda i: (0, 0)),
            pl.BlockSpec((NH, NF), lambda i: (0, 0)),
        ],
        out_specs=[
            pl.BlockSpec((_RB, NF), lambda i: (i, 0)),
            pl.BlockSpec((_RB, ND), lambda i: (i, 0)),
        ],
        out_shape=[
            jax.ShapeDtypeStruct((N, NF), jnp.float32),
            jax.ShapeDtypeStruct((N, ND), jnp.float32),
        ],
    )(h2a, h2b, wg0, wg1, w1t, b1r, w2t)


@jax.jit
def kernel(x, edge_index, edge_attr, W_gc, W1, b1, W2):
    x0 = x[:, :F]
    x1 = x[:, F:]
    row = edge_index[0]
    col = edge_index[1]
    pad = E_PAD - E
    padi = (jnp.arange(pad, dtype=jnp.int32) * 97) % N
    rowp = jnp.concatenate([row, padi])
    colp = jnp.concatenate([col, padi])
    rcp = ((rowp << 14) | colp).reshape(NS, NB, B)
    wp = jnp.concatenate(
        [edge_attr, jnp.zeros((pad,), jnp.float32)]).reshape(NS, NB, B)

    h1a, h1b, h2a, h2b = _sc_prop(x0, x1, rcp, wp)
    h2a = h2a[:N]
    h2b = h2b[:N]

    wgt = W_gc.T
    y, emb = _mlp(h2a, h2b, wgt[:F], wgt[F:], W1.T, b1.reshape(1, NH), W2.T)
    return (y, emb)


# speedup vs baseline: 1.3859x; 1.3859x over previous
import functools

import jax
import jax.numpy as jnp
from jax import lax
from jax.experimental import pallas as pl
from jax.experimental.pallas import tpu as pltpu
from jax.experimental.pallas import tpu_sc as plsc

N = 10000
E = 320000
NF = 128
NH = 256
ND = 128
F = 64
NC = 2
NS = 16
LANES = 16
B = 128
NB = 160
EPT = NB * B
E_PAD = NS * EPT
NPAD = 10240
NSLOT = 4
import numpy as np

MAGIC = np.int32(0x5F3759DF)
CMASK = np.int32(16383)


def _rsqrt16(v):
    i = plsc.bitcast(v, jnp.int32)
    y = plsc.bitcast(MAGIC - (i >> 1), jnp.float32)
    for _ in range(3):
        y = y * (1.5 - 0.5 * v * y * y)
    return jnp.where(v > 0.0, y, 0.0)


def _zero16():
    return jnp.zeros((LANES,), jnp.float32)


def _sc_body(x0, x1, rcb, wb,
             h1a, h1b, h2a, h2b,
             rc_ref, w_ref, dd_ref, abuf, tbuf, rbuf, ridx, cidx,
             dis_sh, h_acc, gsem, ssem):
    c = lax.axis_index("c")
    t = lax.axis_index("s")

    pltpu.sync_copy(rcb.at[t], rc_ref)
    pltpu.sync_copy(wb.at[t], w_ref)

    @pl.loop(0, 160)
    def _(r):
        for j in range(F // LANES):
            dd_ref[r, pl.ds(j * LANES, LANES)] = _zero16()

    for r in range(10):
        for j in range(F // LANES):
            abuf[r, pl.ds(j * LANES, LANES)] = _zero16()

    @pl.loop(0, NB)
    def _(b):
        for k in range(B // LANES):
            sl = pl.ds(k * LANES, LANES)
            cv = rc_ref[b, sl] & CMASK
            plsc.addupdate_scatter(dd_ref, [cv >> 6, cv & 63], w_ref[b, sl])

    pltpu.sync_copy(dd_ref, h_acc.at[pl.ds(160 * t, 160)])
    plsc.subcore_barrier()

    for j in range(NS):
        pltpu.sync_copy(h_acc.at[pl.ds(j * 160 + 10 * t, 10)], tbuf)
        for r in range(10):
            for jj in range(F // LANES):
                sl = pl.ds(jj * LANES, LANES)
                abuf[r, sl] = abuf[r, sl] + tbuf[r, sl]
    for r in range(10):
        for jj in range(F // LANES):
            sl = pl.ds(jj * LANES, LANES)
            abuf[r, sl] = _rsqrt16(abuf[r, sl])
    pltpu.sync_copy(abuf, dis_sh.at[pl.ds(10 * t, 10)])
    plsc.subcore_barrier()
    pltpu.sync_copy(dis_sh, dd_ref)

    @pl.loop(0, NB)
    def _(b):
        for k in range(B // LANES):
            sl = pl.ds(k * LANES, LANES)
            pk = rc_ref[b, sl]
            rv = pk >> 14
            cv = pk & CMASK
            wv = w_ref[b, sl]
            dr = plsc.load_gather(dd_ref, [rv >> 6, rv & 63])
            dc = plsc.load_gather(dd_ref, [cv >> 6, cv & 63])
            w_ref[b, sl] = dr * wv * dc

    def unpack(b, s):
        for k in range(B // LANES):
            sl = pl.ds(k * LANES, LANES)
            pk = rc_ref[b, sl]
            ridx[s, sl] = pk >> 14
            cidx[s, sl] = pk & CMASK

    def hop(src_hbm, dst_hbm):
        plsc.subcore_barrier()

        z0 = 3 * B

        @pl.loop(0, B)
        def _(r):
            for j in range(F // LANES):
                rbuf[z0 + r, pl.ds(j * LANES, LANES)] = _zero16()

        for z in range(5):
            pltpu.sync_copy(rbuf.at[pl.ds(z0, B)],
                            h_acc.at[pl.ds(t * 640 + z * B, B)])
        plsc.subcore_barrier()

        def scale(s, b):
            buf = rbuf.at[pl.ds(s * B, B)]

            @pl.loop(0, B // LANES)
            def _(g):
                nv = w_ref[b, pl.ds(g * LANES, LANES)]
                for i in range(LANES):
                    sv = nv[i]
                    e = g * LANES + i
                    for j in range(F // LANES):
                        sl = pl.ds(j * LANES, LANES)
                        buf[e, sl] = buf[e, sl] * sv

        for b0 in range(2):
            unpack(b0, b0)
            pltpu.async_copy(src_hbm.at[ridx.at[b0]],
                             rbuf.at[pl.ds(b0 * B, B)], gsem.at[b0])

        @pl.loop(0, NB // NSLOT)
        def _(i):
            b4 = i * NSLOT
            for k in range(NSLOT):
                b = b4 + k
                pltpu.make_async_copy(src_hbm.at[pl.ds(0, B)],
                                      rbuf.at[pl.ds(0, B)],
                                      gsem.at[k]).wait()
                scale(k, b)
                pltpu.async_copy(rbuf.at[pl.ds(k * B, B)],
                                 h_acc.at[cidx.at[k]], ssem.at[k], add=True)
                s2 = (k + 2) % NSLOT

                @pl.when(b + 2 < NB)
                def _():
                    @pl.when(b >= 2)
                    def _():
                        pltpu.make_async_copy(
                            rbuf.at[pl.ds(0, B)], h_acc.at[pl.ds(0, B)],
                            ssem.at[s2]).wait()

                    unpack(b + 2, s2)
                    pltpu.async_copy(src_hbm.at[ridx.at[s2]],
                                     rbuf.at[pl.ds(s2 * B, B)], gsem.at[s2])

        for b0 in range(NB - 4, NB):
            pltpu.make_async_copy(rbuf.at[pl.ds(0, B)],
                                  h_acc.at[pl.ds(0, B)],
                                  ssem.at[b0 % NSLOT]).wait()

        plsc.subcore_barrier()
        for z in range(5):
            base = t * 640 + z * B
            pltpu.sync_copy(h_acc.at[pl.ds(base, B)], rbuf.at[pl.ds(0, B)])
            pltpu.sync_copy(rbuf.at[pl.ds(0, B)], dst_hbm.at[pl.ds(base, B)])
        plsc.subcore_barrier()

    @pl.when(c == 0)
    def _():
        hop(x0, h1a)
        hop(h1a, h2a)

    @pl.when(c == 1)
    def _():
        hop(x1, h1b)
        hop(h1b, h2b)


_sc_prop = pl.kernel(
    _sc_body,
    out_type=[jax.ShapeDtypeStruct((NPAD, F), jnp.float32)] * 4,
    mesh=plsc.VectorSubcoreMesh(
        core_axis_name="c", subcore_axis_name="s", num_cores=NC,
        num_subcores=NS),
    compiler_params=pltpu.CompilerParams(
        needs_layout_passes=False, use_tc_tiling_on_sc=False),
    scratch_types=[
        pltpu.VMEM((NB, B), jnp.int32),
        pltpu.VMEM((NB, B), jnp.float32),
        pltpu.VMEM((160, F), jnp.float32),
        pltpu.VMEM((10, F), jnp.float32),
        pltpu.VMEM((10, F), jnp.float32),
        pltpu.VMEM((NSLOT * B, F), jnp.float32),
        pltpu.VMEM((NSLOT, B), jnp.int32),
        pltpu.VMEM((NSLOT, B), jnp.int32),
        pltpu.VMEM_SHARED((160, F), jnp.float32),
        pltpu.VMEM_SHARED((NPAD, F), jnp.float32),
        pltpu.SemaphoreType.DMA((NSLOT,)),
        pltpu.SemaphoreType.DMA((NSLOT,)),
    ],
)


def _mlp_body(h2a_ref, h2b_ref, wg0_ref, wg1_ref, w1t_ref, b1_ref, w2t_ref,
              y_ref, emb_ref):
    a = h2a_ref[...]
    b = h2b_ref[...]
    emb = (jnp.dot(a, wg0_ref[...], preferred_element_type=jnp.float32)
           + jnp.dot(b, wg1_ref[...], preferred_element_type=jnp.float32))
    emb_ref[...] = emb
    u = jnp.maximum(emb, 0.0)
    u = jnp.dot(u, w1t_ref[...], preferred_element_type=jnp.float32)
    u = jnp.maximum(u + b1_ref[...], 0.0)
    y_ref[...] = jnp.dot(u, w2t_ref[...], preferred_element_type=jnp.float32)


_RB = 1000


def _mlp(h2a, h2b, wg0, wg1, w1t, b1r, w2t):
    return pl.pallas_call(
        _mlp_body,
        grid=(N // _RB,),
        in_specs=[
            pl.BlockSpec((_RB, F), lambda i: (i, 0)),
            pl.BlockSpec((_RB, F), lambda i: (i, 0)),
            pl.BlockSpec((F, ND), lambda i: (0, 0)),
            pl.BlockSpec((F, ND), lambda i: (0, 0)),
            pl.BlockSpec((ND, NH), lambda i: (0, 0)),
            pl.BlockSpec((1, NH), lambda i: (0, 0)),
            pl.BlockSpec((NH, NF), lambda i: (0, 0)),
        ],
        out_specs=[
            pl.BlockSpec((_RB, NF), lambda i: (i, 0)),
            pl.BlockSpec((_RB, ND), lambda i: (i, 0)),
        ],
        out_shape=[
            jax.ShapeDtypeStruct((N, NF), jnp.float32),
            jax.ShapeDtypeStruct((N, ND), jnp.float32),
        ],
    )(h2a, h2b, wg0, wg1, w1t, b1r, w2t)


@jax.jit
def kernel(x, edge_index, edge_attr, W_gc, W1, b1, W2):
    x0 = x[:, :F]
    x1 = x[:, F:]
    row = edge_index[0]
    col = edge_index[1]
    pad = E_PAD - E
    padi = (jnp.arange(pad, dtype=jnp.int32) * 97) % N
    rowp = jnp.concatenate([row, padi])
    colp = jnp.concatenate([col, padi])
    rcp = ((rowp << 14) | colp).reshape(NS, NB, B)
    wp = jnp.concatenate(
        [edge_attr, jnp.zeros((pad,), jnp.float32)]).reshape(NS, NB, B)

    h1a, h1b, h2a, h2b = _sc_prop(x0, x1, rcp, wp)
    h2a = h2a[:N]
    h2b = h2b[:N]

    wgt = W_gc.T
    y, emb = _mlp(h2a, h2b, wgt[:F], wgt[F:], W1.T, b1.reshape(1, NH), W2.T)
    return (y, emb)
